# baseline (speedup 1.0000x reference)
import functools

import jax
import jax.numpy as jnp
import numpy as np
from jax.experimental import pallas as pl
from jax.experimental.pallas import tpu as pltpu

_K_PERCENT = 0.1
_LAMBDA_CLASS = 0.1
_ALPHA = 0.5
_BETA = 0.5
_LABEL_SMOOTH = 0.1
_EPS = 1e-7
_INT_MIN = np.int32(-2147483648)


def _row_kernel(x_ref, t_ref, sp_ref, spt_ref, st_ref, bce_ref, tk_ref, *, k):
    x = x_ref[...]
    t = t_ref[...]
    r, hw = x.shape

    def rowsum(v):
        return jnp.sum(v, axis=1, keepdims=True)

    e = jnp.exp(-jnp.abs(x))
    sig = jnp.where(x >= 0.0, 1.0, e) / (1.0 + e)
    s_p = rowsum(sig)
    s_pt = rowsum(sig * t)
    s_t = rowsum(t)
    ts = t * (1.0 - _LABEL_SMOOTH) + 0.5 * _LABEL_SMOOTH
    bce = rowsum(jnp.maximum(x, 0.0) - x * ts + jnp.log1p(e))

    ix = jax.lax.bitcast_convert_type(x, jnp.int32)
    skey = jnp.where(ix < 0, jnp.bitwise_xor(jnp.bitwise_not(ix), _INT_MIN),
                     ix)

    kf = float(k)

    def encode(tf):
        ti = jax.lax.bitcast_convert_type(tf, jnp.int32)
        return jnp.where(ti < 0,
                         jnp.bitwise_xor(jnp.bitwise_not(ti), _INT_MIN), ti)

    def body_a(i, state):
        lo, hi, clo, chi, pinned, psc, dpin = state
        span = hi - lo
        t = hi - span * ((kf - chi) / jnp.maximum(clo - chi, 1.0))
        mid = lo + 0.5 * span
        t = jnp.where(jnp.logical_and(t > lo, t < hi), t, mid)
        st = encode(t)
        c = rowsum(jnp.where(skey >= st, 1.0, 0.0))
        d = c - kf
        newly = jnp.where(jnp.abs(d) <= 1.0, 1 - pinned, 0)
        return (jnp.where(c > kf, t, lo),
                jnp.where(c < kf, t, hi),
                jnp.where(c > kf, c, clo),
                jnp.where(c < kf, c, chi),
                pinned + newly,
                jnp.where(newly > 0, st, psc),
                jnp.where(newly > 0, d, dpin))

    rmin = jnp.min(x, axis=1, keepdims=True)
    rmax = jnp.max(x, axis=1, keepdims=True)

    ns = (hw // 8 // 128) * 128
    if ns == 0:
        ns = hw
    sub_scale = float(hw) / ns
    skey_s = skey[:, :ns]

    def body_a0(j, s):
        lo, hi, clo, chi = s
        span = hi - lo
        t = hi - span * ((kf - chi) / jnp.maximum(clo - chi, 1.0))
        mid = lo + 0.5 * span
        t = jnp.where(jnp.logical_and(t > lo, t < hi), t, mid)
        st = encode(t)
        cs = jnp.sum(jnp.where(skey_s >= st, 1.0, 0.0), axis=1,
                     keepdims=True) * sub_scale
        return (jnp.where(cs > 1.3 * kf, t, lo),
                jnp.where(cs < 0.7 * kf, t, hi),
                jnp.where(cs > 1.3 * kf, cs, clo),
                jnp.where(cs < 0.7 * kf, cs, chi))

    lo0, hi0, clo0, chi0 = jax.lax.fori_loop(
        0, 6, body_a0,
        (rmin, rmax, jnp.full((r, 1), float(hw), jnp.float32),
         jnp.ones((r, 1), jnp.float32)))

    _, _, _, _, pinned0, psc0, dpin = jax.lax.fori_loop(
        0, 8, body_a,
        (lo0, hi0, clo0, chi0,
         jnp.zeros((r, 1), jnp.int32), jnp.zeros((r, 1), jnp.int32),
         jnp.zeros((r, 1), jnp.float32)))

    def cond(state):
        i, _, pinned, _ = state
        return jnp.logical_and(i < 32, jnp.min(pinned) < 1)

    def body(state):
        i, p, pinned, psc = state
        bit = jnp.left_shift(jnp.int32(1), jnp.int32(31) - i)
        cand = jnp.bitwise_or(p, bit)
        scand = jnp.bitwise_xor(cand, _INT_MIN)
        cnt = rowsum(jnp.where(skey >= scand, 1.0, 0.0))
        newly = jnp.where(cnt == float(k), 1 - pinned, 0)
        return (i + 1,
                jnp.where(cnt >= float(k), cand, p),
                pinned + newly,
                jnp.where(newly > 0, scand, psc))

    _, p, pinned, psc = jax.lax.while_loop(
        cond, body,
        (jnp.int32(0), jnp.zeros((r, 1), jnp.int32), pinned0, psc0))

    was_pinned = pinned > 0
    sp_thr = jnp.where(was_pinned, psc, jnp.bitwise_xor(p, _INT_MIN))
    mask_ge = skey >= sp_thr
    c_ge = rowsum(jnp.where(mask_ge, 1.0, 0.0))
    s_ge = rowsum(jnp.where(mask_ge, x, 0.0))
    v_min_in = jnp.min(jnp.where(mask_ge, x, jnp.float32(np.inf)),
                       axis=1, keepdims=True)
    v_max_out = jnp.max(jnp.where(mask_ge, jnp.float32(-np.inf), x),
                        axis=1, keepdims=True)
    adj = jnp.where(dpin > 0.5, -v_min_in,
                    jnp.where(dpin < -0.5, v_max_out, 0.0))
    thr_bits = jnp.where(p < 0, jnp.bitwise_and(p, jnp.int32(2147483647)),
                         jnp.bitwise_not(p))
    x_thr = jax.lax.bitcast_convert_type(thr_bits, jnp.float32)
    tk_sum = jnp.where(was_pinned, s_ge + adj,
                       s_ge - (c_ge - float(k)) * x_thr)
    tk_mean = tk_sum * (1.0 / k)

    lanes = sp_ref.shape[-1]
    for ref, val in ((sp_ref, s_p), (spt_ref, s_pt), (st_ref, s_t),
                     (bce_ref, bce), (tk_ref, tk_mean)):
        ref[...] = jnp.broadcast_to(val[None], (1, r, lanes))


def _finalize_kernel(sp_ref, spt_ref, st_ref, bce_ref, tk_ref, tc_ref,
                     scale_ref, out_ref, *, n_elems):
    sp = jnp.max(sp_ref[...], axis=-1)
    spt = jnp.max(spt_ref[...], axis=-1)
    st = jnp.max(st_ref[...], axis=-1)
    bce_rows = jnp.max(bce_ref[...], axis=-1)
    tk = jnp.max(tk_ref[...], axis=-1)

    b, c = sp.shape
    tp = jnp.sum(spt, axis=0, keepdims=True)
    s_all = jnp.sum(sp, axis=0, keepdims=True)
    t_all = jnp.sum(st, axis=0, keepdims=True)
    fp = s_all - tp
    fn = t_all - tp
    den = tp + _ALPHA * fp + _BETA * fn
    score = tp / jnp.maximum(den, _EPS)
    seg_loss = jnp.sum((1.0 - score) * (t_all > 0.0).astype(jnp.float32),
                       keepdims=True) / c

    bce_loss = jnp.sum(bce_rows, keepdims=True) / n_elems

    peak = tk * scale_ref[0, 0]
    mx = jnp.max(peak, axis=1, keepdims=True)
    lse = mx + jnp.log(jnp.sum(jnp.exp(peak - mx), axis=1, keepdims=True))
    lane = jax.lax.broadcasted_iota(jnp.int32, (b, c), 1)
    sel = jnp.sum(jnp.where(lane == tc_ref[...], peak, 0.0), axis=1,
                  keepdims=True)
    cls_loss = jnp.sum(lse - sel, keepdims=True) / b

    out_ref[...] = seg_loss + _LAMBDA_CLASS * cls_loss + bce_loss


def kernel(inputs, targets_mask, targets_class, scale):
    B, C, H, W = inputs.shape
    HW = H * W
    R = 32 if C % 32 == 0 else (16 if C % 16 == 0 else 8)
    rows = B * C
    cb = C // R
    k = max(1, int(HW * _K_PERCENT))

    x2 = inputs.reshape(rows, HW)
    t2 = targets_mask.reshape(rows, HW)

    stats = pl.pallas_call(
        functools.partial(_row_kernel, k=k),
        grid=(rows // R,),
        in_specs=[
            pl.BlockSpec((R, HW), lambda g: (g, 0)),
            pl.BlockSpec((R, HW), lambda g: (g, 0)),
        ],
        out_specs=[
            pl.BlockSpec((1, R, 128), lambda g: (g // cb, g % cb, 0))
        ] * 5,
        out_shape=[jax.ShapeDtypeStruct((B, C, 128), jnp.float32)] * 5,
        compiler_params=pltpu.CompilerParams(
            dimension_semantics=("parallel",)),
    )(x2, t2)

    tc2 = targets_class.astype(jnp.int32).reshape(B, 1)
    scale2 = scale.astype(jnp.float32).reshape(1, 1)

    out = pl.pallas_call(
        functools.partial(_finalize_kernel, n_elems=float(rows) * HW),
        in_specs=[pl.BlockSpec((B, C, 128), lambda: (0, 0, 0))] * 5
        + [pl.BlockSpec((B, 1), lambda: (0, 0)),
           pl.BlockSpec((1, 1), lambda: (0, 0))],
        out_specs=pl.BlockSpec((1, 1), lambda: (0, 0)),
        out_shape=jax.ShapeDtypeStruct((1, 1), jnp.float32),
    )(*stats, tc2, scale2)

    return out.reshape(())


# speedup vs baseline: 1.0179x; 1.0179x over previous
import functools

import jax
import jax.numpy as jnp
import numpy as np
from jax.experimental import pallas as pl
from jax.experimental.pallas import tpu as pltpu

_K_PERCENT = 0.1
_LAMBDA_CLASS = 0.1
_ALPHA = 0.5
_BETA = 0.5
_LABEL_SMOOTH = 0.1
_EPS = 1e-7
_INT_MIN = np.int32(-2147483648)


def _row_kernel(x_ref, t_ref, sp_ref, spt_ref, st_ref, bce_ref, tk_ref, *, k):
    x = x_ref[...]
    t = t_ref[...]
    r, hw = x.shape

    def rowsum(v):
        return jnp.sum(v, axis=1, keepdims=True)

    e = jnp.exp(-jnp.abs(x))
    sig = jnp.where(x >= 0.0, 1.0, e) / (1.0 + e)
    s_p = rowsum(sig)
    s_pt = rowsum(sig * t)
    s_t = rowsum(t)
    ts = t * (1.0 - _LABEL_SMOOTH) + 0.5 * _LABEL_SMOOTH
    bce = rowsum(jnp.maximum(x, 0.0) - x * ts + jnp.log1p(e))

    ix = jax.lax.bitcast_convert_type(x, jnp.int32)
    skey = jnp.where(ix < 0, jnp.bitwise_xor(jnp.bitwise_not(ix), _INT_MIN),
                     ix)

    kf = float(k)

    def encode(tf):
        ti = jax.lax.bitcast_convert_type(tf, jnp.int32)
        return jnp.where(ti < 0,
                         jnp.bitwise_xor(jnp.bitwise_not(ti), _INT_MIN), ti)

    def cond_a(state):
        i = state[0]
        return jnp.logical_and(i < 16, jnp.min(state[5]) < 1)

    def body_a(state):
        i, lo, hi, clo, chi, pinned, psc, dpin = state
        span = hi - lo
        t = hi - span * ((kf - chi) / jnp.maximum(clo - chi, 1.0))
        mid = lo + 0.5 * span
        t = jnp.where(jnp.logical_and(t > lo, t < hi), t, mid)
        st = encode(t)
        c = rowsum(jnp.where(skey >= st, 1.0, 0.0))
        d = c - kf
        newly = jnp.where(jnp.abs(d) <= 1.0, 1 - pinned, 0)
        return (i + 1,
                jnp.where(c > kf, t, lo),
                jnp.where(c < kf, t, hi),
                jnp.where(c > kf, c, clo),
                jnp.where(c < kf, c, chi),
                pinned + newly,
                jnp.where(newly > 0, st, psc),
                jnp.where(newly > 0, d, dpin))

    rmin = jnp.min(x, axis=1, keepdims=True)
    rmax = jnp.max(x, axis=1, keepdims=True)

    ns = (hw // 8 // 128) * 128
    if ns == 0:
        ns = hw
    sub_scale = float(hw) / ns
    skey_s = skey[:, :ns]

    def body_a0(j, s):
        lo, hi, clo, chi = s
        span = hi - lo
        t = hi - span * ((kf - chi) / jnp.maximum(clo - chi, 1.0))
        mid = lo + 0.5 * span
        t = jnp.where(jnp.logical_and(t > lo, t < hi), t, mid)
        st = encode(t)
        cs = jnp.sum(jnp.where(skey_s >= st, 1.0, 0.0), axis=1,
                     keepdims=True) * sub_scale
        return (jnp.where(cs > 1.3 * kf, t, lo),
                jnp.where(cs < 0.7 * kf, t, hi),
                jnp.where(cs > 1.3 * kf, cs, clo),
                jnp.where(cs < 0.7 * kf, cs, chi))

    lo0, hi0, clo0, chi0 = jax.lax.fori_loop(
        0, 8, body_a0,
        (rmin, rmax, jnp.full((r, 1), float(hw), jnp.float32),
         jnp.ones((r, 1), jnp.float32)))

    _, _, _, _, _, pinned0, psc0, dpin = jax.lax.while_loop(
        cond_a, body_a,
        (jnp.int32(0), lo0, hi0, clo0, chi0,
         jnp.zeros((r, 1), jnp.int32), jnp.zeros((r, 1), jnp.int32),
         jnp.zeros((r, 1), jnp.float32)))

    def cond(state):
        i, _, pinned, _ = state
        return jnp.logical_and(i < 32, jnp.min(pinned) < 1)

    def body(state):
        i, p, pinned, psc = state
        bit = jnp.left_shift(jnp.int32(1), jnp.int32(31) - i)
        cand = jnp.bitwise_or(p, bit)
        scand = jnp.bitwise_xor(cand, _INT_MIN)
        cnt = rowsum(jnp.where(skey >= scand, 1.0, 0.0))
        newly = jnp.where(cnt == float(k), 1 - pinned, 0)
        return (i + 1,
                jnp.where(cnt >= float(k), cand, p),
                pinned + newly,
                jnp.where(newly > 0, scand, psc))

    _, p, pinned, psc = jax.lax.while_loop(
        cond, body,
        (jnp.int32(0), jnp.zeros((r, 1), jnp.int32), pinned0, psc0))

    was_pinned = pinned > 0
    sp_thr = jnp.where(was_pinned, psc, jnp.bitwise_xor(p, _INT_MIN))
    mask_ge = skey >= sp_thr
    c_ge = rowsum(jnp.where(mask_ge, 1.0, 0.0))
    s_ge = rowsum(jnp.where(mask_ge, x, 0.0))
    v_min_in = jnp.min(jnp.where(mask_ge, x, jnp.float32(np.inf)),
                       axis=1, keepdims=True)
    v_max_out = jnp.max(jnp.where(mask_ge, jnp.float32(-np.inf), x),
                        axis=1, keepdims=True)
    adj = jnp.where(dpin > 0.5, -v_min_in,
                    jnp.where(dpin < -0.5, v_max_out, 0.0))
    thr_bits = jnp.where(p < 0, jnp.bitwise_and(p, jnp.int32(2147483647)),
                         jnp.bitwise_not(p))
    x_thr = jax.lax.bitcast_convert_type(thr_bits, jnp.float32)
    tk_sum = jnp.where(was_pinned, s_ge + adj,
                       s_ge - (c_ge - float(k)) * x_thr)
    tk_mean = tk_sum * (1.0 / k)

    lanes = sp_ref.shape[-1]
    for ref, val in ((sp_ref, s_p), (spt_ref, s_pt), (st_ref, s_t),
                     (bce_ref, bce), (tk_ref, tk_mean)):
        ref[...] = jnp.broadcast_to(val[None], (1, r, lanes))


def _finalize_kernel(sp_ref, spt_ref, st_ref, bce_ref, tk_ref, tc_ref,
                     scale_ref, out_ref, *, n_elems):
    sp = jnp.max(sp_ref[...], axis=-1)
    spt = jnp.max(spt_ref[...], axis=-1)
    st = jnp.max(st_ref[...], axis=-1)
    bce_rows = jnp.max(bce_ref[...], axis=-1)
    tk = jnp.max(tk_ref[...], axis=-1)

    b, c = sp.shape
    tp = jnp.sum(spt, axis=0, keepdims=True)
    s_all = jnp.sum(sp, axis=0, keepdims=True)
    t_all = jnp.sum(st, axis=0, keepdims=True)
    fp = s_all - tp
    fn = t_all - tp
    den = tp + _ALPHA * fp + _BETA * fn
    score = tp / jnp.maximum(den, _EPS)
    seg_loss = jnp.sum((1.0 - score) * (t_all > 0.0).astype(jnp.float32),
                       keepdims=True) / c

    bce_loss = jnp.sum(bce_rows, keepdims=True) / n_elems

    peak = tk * scale_ref[0, 0]
    mx = jnp.max(peak, axis=1, keepdims=True)
    lse = mx + jnp.log(jnp.sum(jnp.exp(peak - mx), axis=1, keepdims=True))
    lane = jax.lax.broadcasted_iota(jnp.int32, (b, c), 1)
    sel = jnp.sum(jnp.where(lane == tc_ref[...], peak, 0.0), axis=1,
                  keepdims=True)
    cls_loss = jnp.sum(lse - sel, keepdims=True) / b

    out_ref[...] = seg_loss + _LAMBDA_CLASS * cls_loss + bce_loss


def kernel(inputs, targets_mask, targets_class, scale):
    B, C, H, W = inputs.shape
    HW = H * W
    R = 32 if C % 32 == 0 else (16 if C % 16 == 0 else 8)
    rows = B * C
    cb = C // R
    k = max(1, int(HW * _K_PERCENT))

    x2 = inputs.reshape(rows, HW)
    t2 = targets_mask.reshape(rows, HW)

    stats = pl.pallas_call(
        functools.partial(_row_kernel, k=k),
        grid=(rows // R,),
        in_specs=[
            pl.BlockSpec((R, HW), lambda g: (g, 0)),
            pl.BlockSpec((R, HW), lambda g: (g, 0)),
        ],
        out_specs=[
            pl.BlockSpec((1, R, 128), lambda g: (g // cb, g % cb, 0))
        ] * 5,
        out_shape=[jax.ShapeDtypeStruct((B, C, 128), jnp.float32)] * 5,
        compiler_params=pltpu.CompilerParams(
            dimension_semantics=("parallel",)),
    )(x2, t2)

    tc2 = targets_class.astype(jnp.int32).reshape(B, 1)
    scale2 = scale.astype(jnp.float32).reshape(1, 1)

    out = pl.pallas_call(
        functools.partial(_finalize_kernel, n_elems=float(rows) * HW),
        in_specs=[pl.BlockSpec((B, C, 128), lambda: (0, 0, 0))] * 5
        + [pl.BlockSpec((B, 1), lambda: (0, 0)),
           pl.BlockSpec((1, 1), lambda: (0, 0))],
        out_specs=pl.BlockSpec((1, 1), lambda: (0, 0)),
        out_shape=jax.ShapeDtypeStruct((1, 1), jnp.float32),
    )(*stats, tc2, scale2)

    return out.reshape(())


# speedup vs baseline: 1.0301x; 1.0120x over previous
import functools

import jax
import jax.numpy as jnp
import numpy as np
from jax.experimental import pallas as pl
from jax.experimental.pallas import tpu as pltpu

_K_PERCENT = 0.1
_LAMBDA_CLASS = 0.1
_ALPHA = 0.5
_BETA = 0.5
_LABEL_SMOOTH = 0.1
_EPS = 1e-7
_INT_MIN = np.int32(-2147483648)


def _row_kernel(x_ref, t_ref, sp_ref, spt_ref, st_ref, bce_ref, tk_ref, *, k):
    x = x_ref[...]
    t = t_ref[...]
    r, hw = x.shape

    def rowsum(v):
        return jnp.sum(v, axis=1, keepdims=True)

    e = jnp.exp(-jnp.abs(x))
    sig = jnp.where(x >= 0.0, 1.0, e) / (1.0 + e)
    s_p = rowsum(sig)
    s_pt = rowsum(sig * t)
    s_t = rowsum(t)
    ts = t * (1.0 - _LABEL_SMOOTH) + 0.5 * _LABEL_SMOOTH
    bce = rowsum(jnp.maximum(x, 0.0) - x * ts + jnp.log1p(e))

    ix = jax.lax.bitcast_convert_type(x, jnp.int32)
    skey = jnp.where(ix < 0, jnp.bitwise_xor(jnp.bitwise_not(ix), _INT_MIN),
                     ix)

    kf = float(k)

    def encode(tf):
        ti = jax.lax.bitcast_convert_type(tf, jnp.int32)
        return jnp.where(ti < 0,
                         jnp.bitwise_xor(jnp.bitwise_not(ti), _INT_MIN), ti)

    def cond_a(state):
        i = state[0]
        return jnp.logical_and(i < 16, jnp.min(state[5]) < 1)

    def body_a(state):
        i, lo, hi, clo, chi, pinned, psc, dpin = state
        span = hi - lo
        t = hi - span * ((kf - chi) / jnp.maximum(clo - chi, 1.0))
        mid = lo + 0.5 * span
        t = jnp.where(jnp.logical_and(t > lo, t < hi), t, mid)
        st = encode(t)
        c = rowsum(jnp.where(skey >= st, 1.0, 0.0))
        d = c - kf
        newly = jnp.where(jnp.abs(d) <= 1.0, 1 - pinned, 0)
        return (i + 1,
                jnp.where(c > kf, t, lo),
                jnp.where(c < kf, t, hi),
                jnp.where(c > kf, c, clo),
                jnp.where(c < kf, c, chi),
                pinned + newly,
                jnp.where(newly > 0, st, psc),
                jnp.where(newly > 0, d, dpin))

    rmin = jnp.min(x, axis=1, keepdims=True)
    rmax = jnp.max(x, axis=1, keepdims=True)

    ns = (hw // 8 // 128) * 128
    if ns == 0:
        ns = hw
    sub_scale = float(hw) / ns
    skey_s = skey[:, :ns]

    def body_a0(j, s):
        lo, hi, clo, chi = s
        span = hi - lo
        t = hi - span * ((kf - chi) / jnp.maximum(clo - chi, 1.0))
        mid = lo + 0.5 * span
        t = jnp.where(jnp.logical_and(t > lo, t < hi), t, mid)
        st = encode(t)
        cs = jnp.sum(jnp.where(skey_s >= st, 1.0, 0.0), axis=1,
                     keepdims=True) * sub_scale
        return (jnp.where(cs > 1.3 * kf, t, lo),
                jnp.where(cs < 0.7 * kf, t, hi),
                jnp.where(cs > 1.3 * kf, cs, clo),
                jnp.where(cs < 0.7 * kf, cs, chi))

    lo0, hi0, clo0, chi0 = jax.lax.fori_loop(
        0, 6, body_a0,
        (rmin, rmax, jnp.full((r, 1), float(hw), jnp.float32),
         jnp.ones((r, 1), jnp.float32)))

    _, _, _, _, _, pinned0, psc0, dpin = jax.lax.while_loop(
        cond_a, body_a,
        (jnp.int32(0), lo0, hi0, clo0, chi0,
         jnp.zeros((r, 1), jnp.int32), jnp.zeros((r, 1), jnp.int32),
         jnp.zeros((r, 1), jnp.float32)))

    def cond(state):
        i, _, pinned, _ = state
        return jnp.logical_and(i < 32, jnp.min(pinned) < 1)

    def body(state):
        i, p, pinned, psc = state
        bit = jnp.left_shift(jnp.int32(1), jnp.int32(31) - i)
        cand = jnp.bitwise_or(p, bit)
        scand = jnp.bitwise_xor(cand, _INT_MIN)
        cnt = rowsum(jnp.where(skey >= scand, 1.0, 0.0))
        newly = jnp.where(cnt == float(k), 1 - pinned, 0)
        return (i + 1,
                jnp.where(cnt >= float(k), cand, p),
                pinned + newly,
                jnp.where(newly > 0, scand, psc))

    _, p, pinned, psc = jax.lax.while_loop(
        cond, body,
        (jnp.int32(0), jnp.zeros((r, 1), jnp.int32), pinned0, psc0))

    was_pinned = pinned > 0
    sp_thr = jnp.where(was_pinned, psc, jnp.bitwise_xor(p, _INT_MIN))
    mask_ge = skey >= sp_thr
    c_ge = rowsum(jnp.where(mask_ge, 1.0, 0.0))
    s_ge = rowsum(jnp.where(mask_ge, x, 0.0))
    v_min_in = jnp.min(jnp.where(mask_ge, x, jnp.float32(np.inf)),
                       axis=1, keepdims=True)
    v_max_out = jnp.max(jnp.where(mask_ge, jnp.float32(-np.inf), x),
                        axis=1, keepdims=True)
    adj = jnp.where(dpin > 0.5, -v_min_in,
                    jnp.where(dpin < -0.5, v_max_out, 0.0))
    thr_bits = jnp.where(p < 0, jnp.bitwise_and(p, jnp.int32(2147483647)),
                         jnp.bitwise_not(p))
    x_thr = jax.lax.bitcast_convert_type(thr_bits, jnp.float32)
    tk_sum = jnp.where(was_pinned, s_ge + adj,
                       s_ge - (c_ge - float(k)) * x_thr)
    tk_mean = tk_sum * (1.0 / k)

    lanes = sp_ref.shape[-1]
    for ref, val in ((sp_ref, s_p), (spt_ref, s_pt), (st_ref, s_t),
                     (bce_ref, bce), (tk_ref, tk_mean)):
        ref[...] = jnp.broadcast_to(val[None], (1, r, lanes))


def _finalize_kernel(sp_ref, spt_ref, st_ref, bce_ref, tk_ref, tc_ref,
                     scale_ref, out_ref, *, n_elems):
    sp = jnp.max(sp_ref[...], axis=-1)
    spt = jnp.max(spt_ref[...], axis=-1)
    st = jnp.max(st_ref[...], axis=-1)
    bce_rows = jnp.max(bce_ref[...], axis=-1)
    tk = jnp.max(tk_ref[...], axis=-1)

    b, c = sp.shape
    tp = jnp.sum(spt, axis=0, keepdims=True)
    s_all = jnp.sum(sp, axis=0, keepdims=True)
    t_all = jnp.sum(st, axis=0, keepdims=True)
    fp = s_all - tp
    fn = t_all - tp
    den = tp + _ALPHA * fp + _BETA * fn
    score = tp / jnp.maximum(den, _EPS)
    seg_loss = jnp.sum((1.0 - score) * (t_all > 0.0).astype(jnp.float32),
                       keepdims=True) / c

    bce_loss = jnp.sum(bce_rows, keepdims=True) / n_elems

    peak = tk * scale_ref[0, 0]
    mx = jnp.max(peak, axis=1, keepdims=True)
    lse = mx + jnp.log(jnp.sum(jnp.exp(peak - mx), axis=1, keepdims=True))
    lane = jax.lax.broadcasted_iota(jnp.int32, (b, c), 1)
    sel = jnp.sum(jnp.where(lane == tc_ref[...], peak, 0.0), axis=1,
                  keepdims=True)
    cls_loss = jnp.sum(lse - sel, keepdims=True) / b

    out_ref[...] = seg_loss + _LAMBDA_CLASS * cls_loss + bce_loss


def kernel(inputs, targets_mask, targets_class, scale):
    B, C, H, W = inputs.shape
    HW = H * W
    R = 32 if C % 32 == 0 else (16 if C % 16 == 0 else 8)
    rows = B * C
    cb = C // R
    k = max(1, int(HW * _K_PERCENT))

    x2 = inputs.reshape(rows, HW)
    t2 = targets_mask.reshape(rows, HW)

    stats = pl.pallas_call(
        functools.partial(_row_kernel, k=k),
        grid=(rows // R,),
        in_specs=[
            pl.BlockSpec((R, HW), lambda g: (g, 0)),
            pl.BlockSpec((R, HW), lambda g: (g, 0)),
        ],
        out_specs=[
            pl.BlockSpec((1, R, 128), lambda g: (g // cb, g % cb, 0))
        ] * 5,
        out_shape=[jax.ShapeDtypeStruct((B, C, 128), jnp.float32)] * 5,
        compiler_params=pltpu.CompilerParams(
            dimension_semantics=("parallel",)),
    )(x2, t2)

    tc2 = targets_class.astype(jnp.int32).reshape(B, 1)
    scale2 = scale.astype(jnp.float32).reshape(1, 1)

    out = pl.pallas_call(
        functools.partial(_finalize_kernel, n_elems=float(rows) * HW),
        in_specs=[pl.BlockSpec((B, C, 128), lambda: (0, 0, 0))] * 5
        + [pl.BlockSpec((B, 1), lambda: (0, 0)),
           pl.BlockSpec((1, 1), lambda: (0, 0))],
        out_specs=pl.BlockSpec((1, 1), lambda: (0, 0)),
        out_shape=jax.ShapeDtypeStruct((1, 1), jnp.float32),
    )(*stats, tc2, scale2)

    return out.reshape(())
